# Initial kernel scaffold; baseline (speedup 1.0000x reference)
import jax
import jax.numpy as jnp
from jax.experimental import pallas as pl


def kernel(hidden_states, input_ln_w, qkv_w, q_ln_w, k_ln_w, o_w, post_ln_w, router_w, expert_gate_up_w, expert_down_w, shared_gate_w, shared_up_w, shared_down_w):
    raise NotImplementedError("write your pallas kernel here")


# speedup vs baseline: 1.6808x; 1.6808x over previous
import math
import functools

import jax
import jax.numpy as jnp
from jax import lax
from jax.experimental import pallas as pl
from jax.experimental.pallas import tpu as pltpu

H = 768
NH = 12
DH = 64
HALF = DH // 2
E = 16
DFF = 256
EPS = 1e-06
THETA = 10000.0
S = 2048

TSA = 256
TSQ = 512
TSC = 512
TSD = 1024


def _dot_t(a, b):
    return lax.dot_general(a, b, (((1,), (1,)), ((), ())),
                           preferred_element_type=jnp.float32)


def _silu(x):
    return x * (1.0 / (1.0 + jnp.exp(-x)))


def _stage_a_body(x_ref, w_in_ref, qkvw_ref, qln_ref, kln_ref,
                  q_ref, k_ref, v_ref):
    i = pl.program_id(0)
    x = x_ref[...]
    var = jnp.mean(x * x, axis=1, keepdims=True)
    xn = x * lax.rsqrt(var + EPS) * w_in_ref[...]
    qkv = _dot_t(xn, qkvw_ref[...])

    pos = (i * TSA + lax.broadcasted_iota(jnp.int32, (TSA, 1), 0)
           ).astype(jnp.float32)
    j = lax.broadcasted_iota(jnp.int32, (1, HALF), 1).astype(jnp.float32)
    inv_freq = jnp.exp(j * (-math.log(THETA) / HALF))
    ang = pos * inv_freq
    c = jnp.cos(ang)
    s = jnp.sin(ang)

    def norm_rope(mat, w):
        pieces = []
        for h in range(NH):
            xh = mat[:, h * DH:(h + 1) * DH]
            v_ = jnp.mean(xh * xh, axis=1, keepdims=True)
            xh = xh * lax.rsqrt(v_ + EPS) * w
            x1 = xh[:, :HALF]
            x2 = xh[:, HALF:]
            pieces.append(jnp.concatenate([x1 * c - x2 * s,
                                           x1 * s + x2 * c], axis=1))
        return jnp.concatenate(pieces, axis=1)

    q_ref[...] = norm_rope(qkv[:, :H], qln_ref[...])
    k_ref[...] = norm_rope(qkv[:, H:2 * H], kln_ref[...])
    v_ref[...] = qkv[:, 2 * H:]


def _stage_a(x, w_in, qkv_w, q_ln, k_ln):
    n = S // TSA
    return pl.pallas_call(
        _stage_a_body,
        grid=(n,),
        in_specs=[
            pl.BlockSpec((TSA, H), lambda i: (i, 0)),
            pl.BlockSpec((1, H), lambda i: (0, 0)),
            pl.BlockSpec((3 * H, H), lambda i: (0, 0)),
            pl.BlockSpec((1, DH), lambda i: (0, 0)),
            pl.BlockSpec((1, DH), lambda i: (0, 0)),
        ],
        out_specs=[
            pl.BlockSpec((TSA, H), lambda i: (i, 0)),
            pl.BlockSpec((TSA, H), lambda i: (i, 0)),
            pl.BlockSpec((TSA, H), lambda i: (i, 0)),
        ],
        out_shape=[jax.ShapeDtypeStruct((S, H), jnp.float32)] * 3,
    )(x, w_in, qkv_w, q_ln, k_ln)


def _stage_b_body(q_ref, k_ref, v_ref, o_ref):
    qi = pl.program_id(1)
    q = q_ref[...]
    k = k_ref[...]
    v = v_ref[...]
    rows = qi * TSQ + lax.broadcasted_iota(jnp.int32, (TSQ, 1), 0)
    cols = lax.broadcasted_iota(jnp.int32, (1, S), 1)
    mask = cols <= rows
    outs = []
    for sub in range(2):
        qh = q[:, sub * DH:(sub + 1) * DH]
        kh = k[:, sub * DH:(sub + 1) * DH]
        vh = v[:, sub * DH:(sub + 1) * DH]
        sc = _dot_t(qh, kh) * (DH ** -0.5)
        sc = jnp.where(mask, sc, -1e9)
        m = jnp.max(sc, axis=1, keepdims=True)
        p = jnp.exp(sc - m)
        p = p / jnp.sum(p, axis=1, keepdims=True)
        outs.append(lax.dot_general(p, vh, (((1,), (0,)), ((), ())),
                                    preferred_element_type=jnp.float32))
    o_ref[...] = jnp.concatenate(outs, axis=1)


def _stage_b(q, k, v):
    npair = NH // 2
    nq = S // TSQ
    return pl.pallas_call(
        _stage_b_body,
        grid=(npair, nq),
        in_specs=[
            pl.BlockSpec((TSQ, 2 * DH), lambda p, qi: (qi, p)),
            pl.BlockSpec((S, 2 * DH), lambda p, qi: (0, p)),
            pl.BlockSpec((S, 2 * DH), lambda p, qi: (0, p)),
        ],
        out_specs=pl.BlockSpec((TSQ, 2 * DH), lambda p, qi: (qi, p)),
        out_shape=jax.ShapeDtypeStruct((S, H), jnp.float32),
    )(q, k, v)


def _stage_c_body(ctx_ref, hid_ref, ow_ref, postw_ref, routw_ref,
                  sg_ref, su_ref, sd_ref,
                  base_ref, h2_ref, comb_ref):
    ctx = ctx_ref[...]
    attn_out = _dot_t(ctx, ow_ref[...])
    hidden2 = hid_ref[...] + attn_out
    var = jnp.mean(hidden2 * hidden2, axis=1, keepdims=True)
    h2 = hidden2 * lax.rsqrt(var + EPS) * postw_ref[...]
    h2_ref[...] = h2

    logits = _dot_t(h2, routw_ref[...])
    m = jnp.max(logits, axis=1, keepdims=True)
    p = jnp.exp(logits - m)
    p = p / jnp.sum(p, axis=1, keepdims=True)
    topw = jnp.max(p, axis=1, keepdims=True)
    ie = lax.broadcasted_iota(jnp.int32, p.shape, 1)
    ti = jnp.min(jnp.where(p == topw, ie, E), axis=1, keepdims=True)
    comb_ref[...] = jnp.where(ie == ti, topw, 0.0)

    g = _dot_t(h2, sg_ref[...])
    u = _dot_t(h2, su_ref[...])
    shared = _dot_t(_silu(g) * u, sd_ref[...])
    base_ref[...] = hidden2 + shared


def _stage_c(ctx, hidden, o_w, post_w, rout_w, sg, su, sd):
    n = S // TSC
    return pl.pallas_call(
        _stage_c_body,
        grid=(n,),
        in_specs=[
            pl.BlockSpec((TSC, H), lambda i: (i, 0)),
            pl.BlockSpec((TSC, H), lambda i: (i, 0)),
            pl.BlockSpec((H, H), lambda i: (0, 0)),
            pl.BlockSpec((1, H), lambda i: (0, 0)),
            pl.BlockSpec((E, H), lambda i: (0, 0)),
            pl.BlockSpec((DFF, H), lambda i: (0, 0)),
            pl.BlockSpec((DFF, H), lambda i: (0, 0)),
            pl.BlockSpec((H, DFF), lambda i: (0, 0)),
        ],
        out_specs=[
            pl.BlockSpec((TSC, H), lambda i: (i, 0)),
            pl.BlockSpec((TSC, H), lambda i: (i, 0)),
            pl.BlockSpec((TSC, E), lambda i: (i, 0)),
        ],
        out_shape=[
            jax.ShapeDtypeStruct((S, H), jnp.float32),
            jax.ShapeDtypeStruct((S, H), jnp.float32),
            jax.ShapeDtypeStruct((S, E), jnp.float32),
        ],
    )(ctx, hidden, o_w, post_w, rout_w, sg, su, sd)


def _stage_d_body(h2_ref, comb_ref, base_ref, gu_ref, dw_ref, o_ref):
    e = pl.program_id(1)
    h2 = h2_ref[...]
    gup = _dot_t(h2, gu_ref[0])
    g = gup[:, :DFF]
    u = gup[:, DFF:]
    inter = _silu(g) * u
    eo = _dot_t(inter, dw_ref[0])
    ie = lax.broadcasted_iota(jnp.int32, (TSD, E), 1)
    ce = jnp.sum(jnp.where(ie == e, comb_ref[...], 0.0), axis=1,
                 keepdims=True)

    @pl.when(e == 0)
    def _():
        o_ref[...] = base_ref[...] + ce * eo

    @pl.when(e != 0)
    def _():
        o_ref[...] += ce * eo


def _stage_d(h2, comb, base, gu_w, d_w):
    n = S // TSD
    return pl.pallas_call(
        _stage_d_body,
        grid=(n, E),
        in_specs=[
            pl.BlockSpec((TSD, H), lambda t, e: (t, 0)),
            pl.BlockSpec((TSD, E), lambda t, e: (t, 0)),
            pl.BlockSpec((TSD, H), lambda t, e: (t, 0)),
            pl.BlockSpec((1, 2 * DFF, H), lambda t, e: (e, 0, 0)),
            pl.BlockSpec((1, H, DFF), lambda t, e: (e, 0, 0)),
        ],
        out_specs=pl.BlockSpec((TSD, H), lambda t, e: (t, 0)),
        out_shape=jax.ShapeDtypeStruct((S, H), jnp.float32),
    )(h2, comb, base, gu_w, d_w)


def kernel(hidden_states, input_ln_w, qkv_w, q_ln_w, k_ln_w, o_w, post_ln_w,
           router_w, expert_gate_up_w, expert_down_w, shared_gate_w,
           shared_up_w, shared_down_w):
    B = hidden_states.shape[0]
    x = hidden_states.reshape(S, H)
    q, k, v = _stage_a(x, input_ln_w.reshape(1, H), qkv_w,
                       q_ln_w.reshape(1, DH), k_ln_w.reshape(1, DH))
    ctx = _stage_b(q, k, v)
    base, h2, comb = _stage_c(ctx, x, o_w, post_ln_w.reshape(1, H),
                              router_w, shared_gate_w, shared_up_w,
                              shared_down_w)
    out = _stage_d(h2, comb, base, expert_gate_up_w, expert_down_w)
    return out.reshape(B, S, H)
